# baseline (speedup 1.0000x reference)
import functools

import jax
import jax.numpy as jnp
from jax import lax
from jax.experimental import pallas as pl
from jax.experimental.pallas import tpu as pltpu
from jax.experimental.pallas import tpu_sc as plsc

_EMB = 64
_SCALE = _EMB ** (-0.5)
_NW = 32
_LANES = 16
_NBUF = 4
_BLK = 128


def _sc_embed(xT, table_pad, T, B):
    mesh = plsc.VectorSubcoreMesh(core_axis_name="c", subcore_axis_name="s")
    n_main = T - _NBUF

    @functools.partial(
        pl.kernel,
        mesh=mesh,
        compiler_params=pltpu.CompilerParams(needs_layout_passes=False),
        out_type=jax.ShapeDtypeStruct((T, _EMB, B), jnp.float32),
        scratch_types=[pltpu.VMEM((T, _BLK), jnp.int32)]
        + [pltpu.VMEM((_BLK, 128), jnp.float32) for _ in range(_NBUF)]
        + [pltpu.VMEM((_EMB, _BLK), jnp.float32) for _ in range(2)]
        + [pltpu.SemaphoreType.DMA] * (_NBUF + 2),
    )
    def k(x_hbm, table_hbm, out_hbm, idx_v, *bufs_and_sems):
        rows = bufs_and_sems[:_NBUF]
        obufs = bufs_and_sems[_NBUF:_NBUF + 2]
        g_sem = bufs_and_sems[_NBUF + 2:2 * _NBUF + 2]
        o_sem = bufs_and_sems[2 * _NBUF + 2:]
        wid = lax.axis_index("s") * 2 + lax.axis_index("c")
        b0 = wid * _BLK
        pltpu.sync_copy(x_hbm.at[:, pl.ds(b0, _BLK)], idx_v)

        def start_gather(t, b):
            pltpu.async_copy(table_hbm.at[idx_v.at[t]], rows[b], g_sem[b])

        def wait_gather(b):
            pltpu.make_async_copy(table_hbm.at[pl.ds(0, _BLK)],
                                  rows[b], g_sem[b]).wait()

        def transform(b, p):
            @plsc.parallel_loop(0, _EMB, step=1, unroll=2)
            def _col(c):
                cols = jnp.full((_LANES,), 0, jnp.int32) + c
                for g in range(_BLK // _LANES):
                    toks = lax.iota(jnp.int32, _LANES) + (g * _LANES)
                    v = plsc.load_gather(rows[b], [toks, cols])
                    obufs[p][c, pl.ds(g * _LANES, _LANES)] = v * _SCALE

        def start_out(t, p):
            pltpu.async_copy(obufs[p],
                             out_hbm.at[t, :, pl.ds(b0, _BLK)],
                             o_sem[p])

        def wait_out(p):
            pltpu.make_async_copy(obufs[p],
                                  out_hbm.at[0, :, pl.ds(0, _BLK)],
                                  o_sem[p]).wait()

        for b in range(_NBUF):
            start_gather(b, b)
        for t in range(_NBUF):
            b, p = t % _NBUF, t % 2
            wait_gather(b)
            if t >= 2:
                wait_out(p)
            transform(b, p)
            start_out(t, p)
            start_gather(t + _NBUF, b)

        def main_body(g, carry):
            t0 = g * _NBUF
            for i in range(_NBUF):
                t = t0 + i
                b, p = i, i % 2
                wait_gather(b)
                wait_out(p)
                transform(b, p)
                start_out(t, p)
                start_gather(t + _NBUF, b)
            return carry

        lax.fori_loop(1, n_main // _NBUF, main_body, 0)

        for i in range(_NBUF):
            t = n_main + i
            b, p = i, i % 2
            wait_gather(b)
            wait_out(p)
            transform(b, p)
            start_out(t, p)
        for p in range(2):
            wait_out(p)

    return k(xT, table_pad)


def kernel(x, table):
    B, T = x.shape
    xT = x.T.astype(jnp.int32)
    table_pad = jnp.pad(table, ((0, 0), (0, 128 - _EMB)))
    outT = _sc_embed(xT, table_pad, T, B)
    return outT.transpose(2, 0, 1)


# speedup vs baseline: 1.0023x; 1.0023x over previous
import functools

import jax
import jax.numpy as jnp
from jax import lax
from jax.experimental import pallas as pl
from jax.experimental.pallas import tpu as pltpu
from jax.experimental.pallas import tpu_sc as plsc

_EMB = 64
_SCALE = _EMB ** (-0.5)
_NW = 32
_LANES = 16
_NBUF = 4
_BLK = 128


def _sc_embed(xT, table_pad, T, B):
    mesh = plsc.VectorSubcoreMesh(core_axis_name="c", subcore_axis_name="s")
    n_main = T - _NBUF

    @functools.partial(
        pl.kernel,
        mesh=mesh,
        compiler_params=pltpu.CompilerParams(needs_layout_passes=False),
        out_type=jax.ShapeDtypeStruct((T, _EMB, B), jnp.float32),
        scratch_types=[pltpu.VMEM((T, _BLK), jnp.int32)]
        + [pltpu.VMEM((_BLK, 128), jnp.float32) for _ in range(_NBUF)]
        + [pltpu.VMEM((_EMB, _BLK), jnp.float32) for _ in range(2)]
        + [pltpu.SemaphoreType.DMA] * (_NBUF + 2),
    )
    def k(x_hbm, table_hbm, out_hbm, idx_v, *bufs_and_sems):
        rows = bufs_and_sems[:_NBUF]
        obufs = bufs_and_sems[_NBUF:_NBUF + 2]
        g_sem = bufs_and_sems[_NBUF + 2:2 * _NBUF + 2]
        o_sem = bufs_and_sems[2 * _NBUF + 2:]
        wid = lax.axis_index("s") * 2 + lax.axis_index("c")
        b0 = wid * _BLK
        pltpu.sync_copy(x_hbm.at[:, pl.ds(b0, _BLK)], idx_v)

        def start_gather(t, b):
            pltpu.async_copy(table_hbm.at[idx_v.at[t]], rows[b], g_sem[b])

        def wait_gather(b):
            pltpu.make_async_copy(table_hbm.at[pl.ds(0, _BLK)],
                                  rows[b], g_sem[b]).wait()

        def transform(b, p):
            @plsc.parallel_loop(0, _EMB, step=1, unroll=8)
            def _col(c):
                cols = jnp.full((_LANES,), 0, jnp.int32) + c
                for g in range(_BLK // _LANES):
                    toks = lax.iota(jnp.int32, _LANES) + (g * _LANES)
                    v = plsc.load_gather(rows[b], [toks, cols])
                    obufs[p][c, pl.ds(g * _LANES, _LANES)] = v * _SCALE

        def start_out(t, p):
            pltpu.async_copy(obufs[p],
                             out_hbm.at[t, :, pl.ds(b0, _BLK)],
                             o_sem[p])

        def wait_out(p):
            pltpu.make_async_copy(obufs[p],
                                  out_hbm.at[0, :, pl.ds(0, _BLK)],
                                  o_sem[p]).wait()

        for b in range(_NBUF):
            start_gather(b, b)
        for t in range(_NBUF):
            b, p = t % _NBUF, t % 2
            wait_gather(b)
            if t >= 2:
                wait_out(p)
            transform(b, p)
            start_out(t, p)
            start_gather(t + _NBUF, b)

        def main_body(g, carry):
            t0 = g * _NBUF
            for i in range(_NBUF):
                t = t0 + i
                b, p = i, i % 2
                wait_gather(b)
                wait_out(p)
                transform(b, p)
                start_out(t, p)
                start_gather(t + _NBUF, b)
            return carry

        lax.fori_loop(1, n_main // _NBUF, main_body, 0)

        for i in range(_NBUF):
            t = n_main + i
            b, p = i, i % 2
            wait_gather(b)
            wait_out(p)
            transform(b, p)
            start_out(t, p)
        for p in range(2):
            wait_out(p)

    return k(xT, table_pad)


def kernel(x, table):
    B, T = x.shape
    xT = x.T.astype(jnp.int32)
    table_pad = jnp.pad(table, ((0, 0), (0, 128 - _EMB)))
    outT = _sc_embed(xT, table_pad, T, B)
    return outT.transpose(2, 0, 1)


# speedup vs baseline: 1.1802x; 1.1774x over previous
import functools

import jax
import jax.numpy as jnp
from jax import lax
from jax.experimental import pallas as pl
from jax.experimental.pallas import tpu as pltpu
from jax.experimental.pallas import tpu_sc as plsc

_EMB = 64
_SCALE = _EMB ** (-0.5)
_NW = 32
_LANES = 16
_NBUF = 4
_CHUNK = 128


def _sc_embed(x2d, table_pad):
    n_rows = x2d.shape[0]
    n_chunks = n_rows // _NW
    total = n_rows * _CHUNK
    mesh = plsc.VectorSubcoreMesh(core_axis_name="c", subcore_axis_name="s")
    n_main = n_chunks - _NBUF

    @functools.partial(
        pl.kernel,
        mesh=mesh,
        out_type=jax.ShapeDtypeStruct((total, 128), jnp.float32),
        scratch_types=[
            pltpu.VMEM((n_chunks, _CHUNK), jnp.int32),
            pltpu.VMEM((_NBUF, _CHUNK, 128), jnp.float32),
        ]
        + [pltpu.SemaphoreType.DMA] * (2 * _NBUF),
    )
    def k(x_hbm, table_hbm, out_hbm, idx_v, rows_v, *sems):
        g_sem = sems[:_NBUF]
        o_sem = sems[_NBUF:]
        wid = lax.axis_index("s") * 2 + lax.axis_index("c")
        row0 = wid * n_chunks
        pltpu.sync_copy(x_hbm.at[pl.ds(row0, n_chunks)], idx_v)
        out0 = wid * n_chunks * _CHUNK

        def start_gather(c, b):
            pltpu.async_copy(table_hbm.at[idx_v.at[c]], rows_v.at[b],
                             g_sem[b])

        def wait_gather(b):
            pltpu.make_async_copy(table_hbm.at[pl.ds(0, _CHUNK)],
                                  rows_v.at[b], g_sem[b]).wait()

        def scale(b):
            @plsc.parallel_loop(0, _CHUNK, step=1, unroll=4)
            def _scale_row(r):
                for kk in range(_EMB // _LANES):
                    sl = pl.ds(kk * _LANES, _LANES)
                    rows_v[b, r, sl] = rows_v[b, r, sl] * _SCALE

        def start_out(c, b):
            pltpu.async_copy(rows_v.at[b],
                             out_hbm.at[pl.ds(out0 + c * _CHUNK, _CHUNK)],
                             o_sem[b])

        def wait_out(b):
            pltpu.make_async_copy(rows_v.at[b],
                                  out_hbm.at[pl.ds(0, _CHUNK)],
                                  o_sem[b]).wait()

        for b in range(_NBUF):
            start_gather(b, b)

        def main_body(g, carry):
            c0 = g * _NBUF
            for b in range(_NBUF):
                c = c0 + b
                wait_gather(b)
                scale(b)
                start_out(c, b)
                wait_out(b)
                start_gather(c + _NBUF, b)
            return carry

        lax.fori_loop(0, n_main // _NBUF, main_body, 0)

        for b in range(_NBUF):
            c = n_main + b
            wait_gather(b)
            scale(b)
            start_out(c, b)
        for b in range(_NBUF):
            wait_out(b)

    return k(x2d, table_pad)


def kernel(x, table):
    B, T = x.shape
    n_tok = B * T
    x2d = x.reshape(n_tok // _CHUNK, _CHUNK).astype(jnp.int32)
    table_pad = jnp.pad(table, ((0, 0), (0, 128 - _EMB)))
    out = _sc_embed(x2d, table_pad)
    return out[:, :_EMB].reshape(B, T, _EMB)


# speedup vs baseline: 1.1807x; 1.0004x over previous
import functools

import jax
import jax.numpy as jnp
from jax import lax
from jax.experimental import pallas as pl
from jax.experimental.pallas import tpu as pltpu
from jax.experimental.pallas import tpu_sc as plsc

_EMB = 64
_SCALE = _EMB ** (-0.5)
_NW = 32
_LANES = 16
_NBUF = 4
_CHUNK = 128


def _sc_embed(x2d, table_pad):
    n_rows = x2d.shape[0]
    n_chunks = n_rows // _NW
    total = n_rows * _CHUNK
    mesh = plsc.VectorSubcoreMesh(core_axis_name="c", subcore_axis_name="s")
    n_main = n_chunks - _NBUF

    @functools.partial(
        pl.kernel,
        mesh=mesh,
        out_type=jax.ShapeDtypeStruct((total, 128), jnp.float32),
        scratch_types=[
            pltpu.VMEM((n_chunks, _CHUNK), jnp.int32),
            pltpu.VMEM((_NBUF, _CHUNK, 128), jnp.float32),
        ]
        + [pltpu.SemaphoreType.DMA] * (2 * _NBUF),
    )
    def k(x_hbm, table_hbm, out_hbm, idx_v, rows_v, *sems):
        g_sem = sems[:_NBUF]
        o_sem = sems[_NBUF:]
        wid = lax.axis_index("s") * 2 + lax.axis_index("c")
        row0 = wid * n_chunks
        pltpu.sync_copy(x_hbm.at[pl.ds(row0, n_chunks)], idx_v)
        out0 = wid * n_chunks * _CHUNK

        def start_gather(c, b):
            pltpu.async_copy(table_hbm.at[idx_v.at[c]], rows_v.at[b],
                             g_sem[b])

        def wait_gather(b):
            pltpu.make_async_copy(table_hbm.at[pl.ds(0, _CHUNK)],
                                  rows_v.at[b], g_sem[b]).wait()

        def scale(b):
            @plsc.parallel_loop(0, _CHUNK, step=1, unroll=4)
            def _scale_row(r):
                for kk in range(_EMB // _LANES):
                    sl = pl.ds(kk * _LANES, _LANES)
                    rows_v[b, r, sl] = rows_v[b, r, sl] * _SCALE

        def start_out(c, b):
            pltpu.async_copy(rows_v.at[b],
                             out_hbm.at[pl.ds(out0 + c * _CHUNK, _CHUNK)],
                             o_sem[b])

        def wait_out(b):
            pltpu.make_async_copy(rows_v.at[b],
                                  out_hbm.at[pl.ds(0, _CHUNK)],
                                  o_sem[b]).wait()

        for b in range(_NBUF):
            start_gather(b, b)

        def main_body(g, carry):
            c0 = g * _NBUF
            for b in range(_NBUF):
                c = c0 + b
                wait_gather(b)
                scale(b)
                start_out(c, b)
                wait_out(b)
                start_gather(c + _NBUF, b)
            return carry

        lax.fori_loop(0, n_main // _NBUF, main_body, 0)

        for b in range(_NBUF):
            c = n_main + b
            wait_gather(b)
            scale(b)
            start_out(c, b)
        for b in range(_NBUF):
            wait_out(b)

    return k(x2d, table_pad)


def kernel(x, table):
    B, T = x.shape
    n_tok = B * T
    x2d = x.reshape(n_tok // _CHUNK, _CHUNK).astype(jnp.int32)
    table_pad = lax.optimization_barrier(
        jnp.pad(table, ((0, 0), (0, 128 - _EMB))))
    out = _sc_embed(x2d, table_pad)
    return out[:, :_EMB].reshape(B, T, _EMB)
